# baseline (speedup 1.0000x reference)
import functools

import jax
import jax.numpy as jnp
from jax import lax
from jax.experimental import pallas as pl
from jax.experimental.pallas import tpu as pltpu
from jax.experimental.pallas import tpu_sc as plsc

D = 32
NC, NS = 2, 16
NW = NC * NS
CH = 128

_MESH = dict(core_axis_name="c", subcore_axis_name="s")
_SC_PARAMS = pltpu.CompilerParams(use_tc_tiling_on_sc=False)


def _slabs(n):
    base = ((n // NS) // 8) * 8
    slabs = [base] * NS
    slabs[-1] = n - base * (NS - 1)
    starts = [base * i for i in range(NS)]
    return starts, slabs


def _sc_gather(table, idx):
    n, d = table.shape
    e = idx.shape[0]
    nch = e // CH
    nfull, rem = nch // NW, nch % NW
    mesh = plsc.VectorSubcoreMesh(**_MESH)

    @functools.partial(
        pl.kernel,
        out_type=jax.ShapeDtypeStruct((e, d), jnp.float32),
        mesh=mesh,
        compiler_params=_SC_PARAMS,
        scratch_types=[
            pltpu.VMEM((8, CH), jnp.int32),
            pltpu.VMEM((CH, d), jnp.float32),
        ],
    )
    def k(table_hbm, idx_hbm, out_hbm, idx2d, rows):
        cid = lax.axis_index("c")
        sid = lax.axis_index("s")
        w = sid * NC + cid
        nloc = jnp.where(w < rem, nfull + 1, nfull)

        def body(i, carry):
            c = w + i * NW
            pltpu.sync_copy(idx_hbm.at[pl.ds(c * CH, CH)], idx2d.at[0])
            pltpu.sync_copy(table_hbm.at[idx2d.at[0]], rows)
            pltpu.sync_copy(rows, out_hbm.at[pl.ds(c * CH, CH)])
            return carry

        lax.fori_loop(0, nloc, body, 0)

    return k(table, idx)


def _sc_scatter_add(vals, idx, zeros, const_rows=None):
    n = zeros.shape[0]
    e = idx.shape[0]
    nch = e // CH
    nfull, rem = nch // NW, nch % NW
    mesh = plsc.VectorSubcoreMesh(**_MESH)
    starts, sizes = _slabs(n)
    use_const = const_rows is not None
    ins = ((idx, zeros, const_rows) if use_const else (vals, idx, zeros))

    @functools.partial(
        pl.kernel,
        out_type=jax.ShapeDtypeStruct((NC, n, D), jnp.float32),
        mesh=mesh,
        compiler_params=_SC_PARAMS,
        scratch_types=[
            pltpu.VMEM((8, CH), jnp.int32),
            pltpu.VMEM((CH, D), jnp.float32),
            pltpu.VMEM_SHARED((n, D), jnp.float32),
        ],
    )
    def k(*refs):
        if use_const:
            idx_hbm, zeros_hbm, const_hbm, out_hbm, idx2d, vbuf, acc = refs
            vals_hbm = None
        else:
            vals_hbm, idx_hbm, zeros_hbm, out_hbm, idx2d, vbuf, acc = refs
        cid = lax.axis_index("c")
        sid = lax.axis_index("s")
        w = sid * NC + cid

        for s in range(NS):
            @pl.when(sid == s)
            def _():
                pltpu.sync_copy(zeros_hbm.at[pl.ds(starts[s], sizes[s])],
                                acc.at[pl.ds(starts[s], sizes[s])])
        if use_const:
            pltpu.sync_copy(const_hbm, vbuf)
        plsc.subcore_barrier()

        nloc = jnp.where(w < rem, nfull + 1, nfull)

        def body(i, carry):
            c = w + i * NW
            pltpu.sync_copy(idx_hbm.at[pl.ds(c * CH, CH)], idx2d.at[0])
            if not use_const:
                pltpu.sync_copy(vals_hbm.at[pl.ds(c * CH, CH)], vbuf)
            pltpu.sync_copy(vbuf, acc.at[idx2d.at[0]], add=True)
            return carry

        lax.fori_loop(0, nloc, body, 0)
        plsc.subcore_barrier()

        for s in range(NS):
            @pl.when(sid == s)
            def _():
                pltpu.sync_copy(acc.at[pl.ds(starts[s], sizes[s])],
                                out_hbm.at[cid, pl.ds(starts[s], sizes[s])])

    return k(*ins)


def _dot_t(a, b_ref):
    return lax.dot_general(a, b_ref[...], (((1,), (1,)), ((), ())),
                           preferred_element_type=jnp.float32)


def _tc_prep(x, lin0W, lin0b, degp):
    n = x.shape[0]

    def body(x_ref, w_ref, b_ref, degp_ref, out_ref, rdeg_ref):
        out_ref[...] = jnp.maximum(_dot_t(x_ref[...], w_ref) + b_ref[...], 0.0)
        dp = degp_ref[...]
        deg = dp[0, :, 0:1] + dp[1, :, 0:1]
        rdeg_ref[...] = 1.0 / jnp.maximum(deg, 1.0)

    return pl.pallas_call(
        body,
        out_shape=(jax.ShapeDtypeStruct((n, D), jnp.float32),
                   jax.ShapeDtypeStruct((n, 1), jnp.float32)),
    )(x, lin0W, lin0b, degp)


def _tc_msg(edge_attr, t, W1, b1, W2, Emat, Smat):
    e = edge_attr.shape[0]
    EB = 1000
    grid = e // EB

    def body(ea_ref, t_ref, w1_ref, b1_ref, w2_ref, em_ref, sm_ref, msg_ref):
        eh = jnp.maximum(_dot_t(ea_ref[...], w1_ref) + b1_ref[...], 0.0)
        w = lax.dot_general(eh.astype(jnp.bfloat16), w2_ref[...],
                            (((1,), (1,)), ((), ())),
                            preferred_element_type=jnp.float32)
        texp = jnp.dot(t_ref[...].astype(jnp.bfloat16), em_ref[...],
                       preferred_element_type=jnp.float32)
        msg_ref[...] = jnp.dot((texp * w).astype(jnp.bfloat16), sm_ref[...],
                               preferred_element_type=jnp.float32)

    return pl.pallas_call(
        body,
        grid=(grid,),
        in_specs=[
            pl.BlockSpec((EB, 4), lambda i: (i, 0)),
            pl.BlockSpec((EB, D), lambda i: (i, 0)),
            pl.BlockSpec((128, 4), lambda i: (0, 0)),
            pl.BlockSpec((1, 128), lambda i: (0, 0)),
            pl.BlockSpec((D * D, 128), lambda i: (0, 0)),
            pl.BlockSpec((D, D * D), lambda i: (0, 0)),
            pl.BlockSpec((D * D, D), lambda i: (0, 0)),
        ],
        out_specs=pl.BlockSpec((EB, D), lambda i: (i, 0)),
        out_shape=jax.ShapeDtypeStruct((e, D), jnp.float32),
    )(edge_attr, t, W1, b1, W2, Emat, Smat)


def _tc_update(h, aggp, rdeg, rootW, convb, gruWi, gruWh, gbi, gbh):
    n = h.shape[0]

    def body(h_ref, aggp_ref, rdeg_ref, rw_ref, cb_ref, wi_ref, wh_ref,
             bi_ref, bh_ref, out_ref):
        hv = h_ref[...]
        ap = aggp_ref[...]
        agg = (ap[0] + ap[1]) * rdeg_ref[...]
        m = jnp.maximum(jnp.dot(hv, rw_ref[...],
                                preferred_element_type=jnp.float32)
                        + agg + cb_ref[...], 0.0)
        gi = _dot_t(m, wi_ref) + bi_ref[...]
        gh = _dot_t(hv, wh_ref) + bh_ref[...]
        r = jax.nn.sigmoid(gi[:, 0:D] + gh[:, 0:D])
        z = jax.nn.sigmoid(gi[:, D:2 * D] + gh[:, D:2 * D])
        nn = jnp.tanh(gi[:, 2 * D:3 * D] + r * gh[:, 2 * D:3 * D])
        out_ref[...] = (1.0 - z) * nn + z * hv

    return pl.pallas_call(
        body,
        out_shape=jax.ShapeDtypeStruct((n, D), jnp.float32),
    )(h, aggp, rdeg, rootW, convb, gruWi, gruWh, gbi, gbh)


def _tc_final(out, bcol, fp, fc1W, fc1b, bng, bnb, lstmWi, lstmWh, lstmb,
              lin1W, lin1b, lin2W, lin2b):
    n = out.shape[0]
    g = fp.shape[0]
    odim = lin2W.shape[0]

    def body(out_ref, bcol_ref, fp_ref, fc1_ref, fc1b_ref, bng_ref, bnb_ref,
             wi_ref, wh_ref, lb_ref, l1_ref, l1b_ref, l2_ref, l2b_ref,
             res_ref):
        outv = out_ref[...]
        bc = bcol_ref[...]
        gids = lax.broadcasted_iota(jnp.int32, (n, g), 1)
        mask = bc == gids
        hfp = _dot_t(fp_ref[...], fc1_ref) + fc1b_ref[...]
        hfp = hfp * (bng_ref[...] / jnp.sqrt(1.0 + 1e-5)) + bnb_ref[...]
        out_fp = jnp.where(hfp > 0.0, hfp,
                           jnp.exp(jnp.minimum(hfp, 0.0)) - 1.0)
        q_star = jnp.zeros((g, 2 * D), jnp.float32)
        hs = jnp.zeros((g, D), jnp.float32)
        cs = jnp.zeros((g, D), jnp.float32)
        for _ in range(3):
            gg = _dot_t(q_star, wi_ref) + _dot_t(hs, wh_ref) + lb_ref[...]
            i_ = jax.nn.sigmoid(gg[:, 0:D])
            f_ = jax.nn.sigmoid(gg[:, D:2 * D])
            g_ = jnp.tanh(gg[:, 2 * D:3 * D])
            o_ = jax.nn.sigmoid(gg[:, 3 * D:4 * D])
            cs = f_ * cs + i_ * g_
            hs = o_ * jnp.tanh(cs)
            e2 = lax.dot_general(outv, hs, (((1,), (1,)), ((), ())),
                                 preferred_element_type=jnp.float32)
            em = jnp.where(mask, e2, -1e30)
            mseg = jnp.max(em, axis=0, keepdims=True)
            msegc = jnp.where(mseg < -1e29, 0.0, mseg)
            a = jnp.exp(em - msegc)
            den = jnp.maximum(jnp.sum(a, axis=0, keepdims=True), 1e-16)
            an = a / den
            rvec = lax.dot_general(an, outv, (((0,), (0,)), ((), ())),
                                   preferred_element_type=jnp.float32)
            q_star = jnp.concatenate([hs, rvec], axis=1)
        pooled = jnp.maximum(_dot_t(q_star, l1_ref) + l1b_ref[...], 0.0)
        cat = jnp.concatenate([pooled, out_fp], axis=1)
        res_ref[...] = _dot_t(cat, l2_ref) + l2b_ref[...]

    return pl.pallas_call(
        body,
        out_shape=jax.ShapeDtypeStruct((g, odim), jnp.float32),
    )(out, bcol, fp, fc1W, fc1b, bng, bnb, lstmWi, lstmWh, lstmb,
      lin1W, lin1b, lin2W, lin2b)


def kernel(x, fp, edge_attr, params, edge_index, batch):
    p = params
    n = x.shape[0]
    src = edge_index[0]
    dst = edge_index[1]
    zeros = jnp.zeros((n, D), jnp.float32)
    onecol = jnp.zeros((CH, D), jnp.float32).at[:, 0].set(1.0)
    eye = jnp.eye(D, dtype=jnp.float32)
    emat = jnp.repeat(eye, D, axis=1)
    smat = jnp.tile(eye, (D, 1))
    w2_bf = p['enn_W2'].astype(jnp.bfloat16)
    emat_bf = emat.astype(jnp.bfloat16)
    smat_bf = smat.astype(jnp.bfloat16)

    degp = _sc_scatter_add(None, dst, zeros, const_rows=onecol)
    out, rdeg = _tc_prep(x, p['lin0_W'], p['lin0_b'].reshape(1, -1), degp)
    for _ in range(3):
        t = _sc_gather(out, src)
        msg = _tc_msg(edge_attr, t, p['enn_W1'], p['enn_b1'].reshape(1, -1),
                      w2_bf, emat_bf, smat_bf)
        aggp = _sc_scatter_add(msg, dst, zeros)
        out = _tc_update(out, aggp, rdeg, p['root_W'],
                         p['conv_b'].reshape(1, -1), p['gru_Wi'], p['gru_Wh'],
                         p['gru_bi'].reshape(1, -1), p['gru_bh'].reshape(1, -1))
    lstmb = (p['lstm_bi'] + p['lstm_bh']).reshape(1, -1)
    return _tc_final(out, batch.reshape(-1, 1), fp, p['fc1_W'],
                     p['fc1_b'].reshape(1, -1), p['bn1_g'].reshape(1, -1),
                     p['bn1_b'].reshape(1, -1), p['lstm_Wi'], p['lstm_Wh'],
                     lstmb, p['lin1_W'], p['lin1_b'].reshape(1, -1),
                     p['lin2_W'], p['lin2_b'].reshape(1, -1))


# speedup vs baseline: 1.1883x; 1.1883x over previous
import functools

import jax
import jax.numpy as jnp
from jax import lax
from jax.experimental import pallas as pl
from jax.experimental.pallas import tpu as pltpu
from jax.experimental.pallas import tpu_sc as plsc

D = 32
NC, NS = 2, 16
NW = NC * NS
CH = 128

_MESH = dict(core_axis_name="c", subcore_axis_name="s")
_SC_PARAMS = pltpu.CompilerParams(use_tc_tiling_on_sc=False)


def _slabs(n):
    base = ((n // NS) // 8) * 8
    slabs = [base] * NS
    slabs[-1] = n - base * (NS - 1)
    starts = [base * i for i in range(NS)]
    return starts, slabs


def _sc_gather(table, idx):
    n, d = table.shape
    e = idx.shape[0]
    nch = e // CH
    nfull, rem = nch // NW, nch % NW
    mesh = plsc.VectorSubcoreMesh(**_MESH)

    @functools.partial(
        pl.kernel,
        out_type=jax.ShapeDtypeStruct((e, d), jnp.float32),
        mesh=mesh,
        compiler_params=_SC_PARAMS,
        scratch_types=[
            pltpu.VMEM((8, CH), jnp.int32),
            pltpu.VMEM((CH, d), jnp.float32),
        ],
    )
    def k(table_hbm, idx_hbm, out_hbm, idx2d, rows):
        cid = lax.axis_index("c")
        sid = lax.axis_index("s")
        w = sid * NC + cid
        nloc = jnp.where(w < rem, nfull + 1, nfull)

        def body(i, carry):
            c = w + i * NW
            pltpu.sync_copy(idx_hbm.at[pl.ds(c * CH, CH)], idx2d.at[0])
            pltpu.sync_copy(table_hbm.at[idx2d.at[0]], rows)
            pltpu.sync_copy(rows, out_hbm.at[pl.ds(c * CH, CH)])
            return carry

        lax.fori_loop(0, nloc, body, 0)

    return k(table, idx)


def _sc_scatter_add(vals, idx, zeros, const_rows=None):
    n = zeros.shape[0]
    e = idx.shape[0]
    nch = e // CH
    nfull, rem = nch // NW, nch % NW
    mesh = plsc.VectorSubcoreMesh(**_MESH)
    starts, sizes = _slabs(n)
    use_const = const_rows is not None
    ins = ((idx, zeros, const_rows) if use_const else (vals, idx, zeros))

    @functools.partial(
        pl.kernel,
        out_type=jax.ShapeDtypeStruct((NC, n, D), jnp.float32),
        mesh=mesh,
        compiler_params=_SC_PARAMS,
        scratch_types=[
            pltpu.VMEM((8, CH), jnp.int32),
            pltpu.VMEM((CH, D), jnp.float32),
            pltpu.VMEM_SHARED((n, D), jnp.float32),
        ],
    )
    def k(*refs):
        if use_const:
            idx_hbm, zeros_hbm, const_hbm, out_hbm, idx2d, vbuf, acc = refs
            vals_hbm = None
        else:
            vals_hbm, idx_hbm, zeros_hbm, out_hbm, idx2d, vbuf, acc = refs
        cid = lax.axis_index("c")
        sid = lax.axis_index("s")
        w = sid * NC + cid

        for s in range(NS):
            @pl.when(sid == s)
            def _():
                pltpu.sync_copy(zeros_hbm.at[pl.ds(starts[s], sizes[s])],
                                acc.at[pl.ds(starts[s], sizes[s])])
        if use_const:
            pltpu.sync_copy(const_hbm, vbuf)
        plsc.subcore_barrier()

        nloc = jnp.where(w < rem, nfull + 1, nfull)

        def body(i, carry):
            c = w + i * NW
            pltpu.sync_copy(idx_hbm.at[pl.ds(c * CH, CH)], idx2d.at[0])
            if not use_const:
                pltpu.sync_copy(vals_hbm.at[pl.ds(c * CH, CH)], vbuf)
            pltpu.sync_copy(vbuf, acc.at[idx2d.at[0]], add=True)
            return carry

        lax.fori_loop(0, nloc, body, 0)
        plsc.subcore_barrier()

        for s in range(NS):
            @pl.when(sid == s)
            def _():
                pltpu.sync_copy(acc.at[pl.ds(starts[s], sizes[s])],
                                out_hbm.at[cid, pl.ds(starts[s], sizes[s])])

    return k(*ins)


def _dot_t(a, b_ref):
    return lax.dot_general(a, b_ref[...], (((1,), (1,)), ((), ())),
                           preferred_element_type=jnp.float32)


def _tc_prep(x, lin0W, lin0b, degp):
    n = x.shape[0]

    def body(x_ref, w_ref, b_ref, degp_ref, out_ref, rdeg_ref):
        out_ref[...] = jnp.maximum(_dot_t(x_ref[...], w_ref) + b_ref[...], 0.0)
        dp = degp_ref[...]
        deg = dp[0, :, 0:1] + dp[1, :, 0:1]
        rdeg_ref[...] = 1.0 / jnp.maximum(deg, 1.0)

    return pl.pallas_call(
        body,
        out_shape=(jax.ShapeDtypeStruct((n, D), jnp.float32),
                   jax.ShapeDtypeStruct((n, 1), jnp.float32)),
    )(x, lin0W, lin0b, degp)


def _tc_msg(edge_attr, t, W1, b1, W2, Emat, Smat):
    e = edge_attr.shape[0]
    EB = 1000
    grid = e // EB

    def body(ea_ref, t_ref, w1_ref, b1_ref, w2_ref, em_ref, sm_ref, msg_ref):
        eh = jnp.maximum(_dot_t(ea_ref[...], w1_ref) + b1_ref[...], 0.0)
        w = lax.dot_general(eh.astype(jnp.bfloat16), w2_ref[...],
                            (((1,), (1,)), ((), ())),
                            preferred_element_type=jnp.float32)
        texp = jnp.dot(t_ref[...].astype(jnp.bfloat16), em_ref[...],
                       preferred_element_type=jnp.float32)
        p = texp * w
        p = p[:, :512] + p[:, 512:]
        p = p[:, :256] + p[:, 256:]
        p = p[:, :128] + p[:, 128:]
        msg_ref[...] = jnp.dot(p, sm_ref[...],
                               preferred_element_type=jnp.float32)

    return pl.pallas_call(
        body,
        grid=(grid,),
        in_specs=[
            pl.BlockSpec((EB, 4), lambda i: (i, 0)),
            pl.BlockSpec((EB, D), lambda i: (i, 0)),
            pl.BlockSpec((128, 4), lambda i: (0, 0)),
            pl.BlockSpec((1, 128), lambda i: (0, 0)),
            pl.BlockSpec((D * D, 128), lambda i: (0, 0)),
            pl.BlockSpec((D, D * D), lambda i: (0, 0)),
            pl.BlockSpec((4 * D, D), lambda i: (0, 0)),
        ],
        out_specs=pl.BlockSpec((EB, D), lambda i: (i, 0)),
        out_shape=jax.ShapeDtypeStruct((e, D), jnp.float32),
    )(edge_attr, t, W1, b1, W2, Emat, Smat)


def _tc_update(h, aggp, rdeg, rootW, convb, gruWi, gruWh, gbi, gbh):
    n = h.shape[0]

    def body(h_ref, aggp_ref, rdeg_ref, rw_ref, cb_ref, wi_ref, wh_ref,
             bi_ref, bh_ref, out_ref):
        hv = h_ref[...]
        ap = aggp_ref[...]
        agg = (ap[0] + ap[1]) * rdeg_ref[...]
        m = jnp.maximum(jnp.dot(hv, rw_ref[...],
                                preferred_element_type=jnp.float32)
                        + agg + cb_ref[...], 0.0)
        gi = _dot_t(m, wi_ref) + bi_ref[...]
        gh = _dot_t(hv, wh_ref) + bh_ref[...]
        r = jax.nn.sigmoid(gi[:, 0:D] + gh[:, 0:D])
        z = jax.nn.sigmoid(gi[:, D:2 * D] + gh[:, D:2 * D])
        nn = jnp.tanh(gi[:, 2 * D:3 * D] + r * gh[:, 2 * D:3 * D])
        out_ref[...] = (1.0 - z) * nn + z * hv

    return pl.pallas_call(
        body,
        out_shape=jax.ShapeDtypeStruct((n, D), jnp.float32),
    )(h, aggp, rdeg, rootW, convb, gruWi, gruWh, gbi, gbh)


def _tc_final(out, bcol, fp, fc1W, fc1b, bng, bnb, lstmWi, lstmWh, lstmb,
              lin1W, lin1b, lin2W, lin2b):
    n = out.shape[0]
    g = fp.shape[0]
    odim = lin2W.shape[0]

    def body(out_ref, bcol_ref, fp_ref, fc1_ref, fc1b_ref, bng_ref, bnb_ref,
             wi_ref, wh_ref, lb_ref, l1_ref, l1b_ref, l2_ref, l2b_ref,
             res_ref):
        outv = out_ref[...]
        bc = bcol_ref[...]
        gids = lax.broadcasted_iota(jnp.int32, (n, g), 1)
        mask = bc == gids
        hfp = _dot_t(fp_ref[...], fc1_ref) + fc1b_ref[...]
        hfp = hfp * (bng_ref[...] / jnp.sqrt(1.0 + 1e-5)) + bnb_ref[...]
        out_fp = jnp.where(hfp > 0.0, hfp,
                           jnp.exp(jnp.minimum(hfp, 0.0)) - 1.0)
        q_star = jnp.zeros((g, 2 * D), jnp.float32)
        hs = jnp.zeros((g, D), jnp.float32)
        cs = jnp.zeros((g, D), jnp.float32)
        for _ in range(3):
            gg = _dot_t(q_star, wi_ref) + _dot_t(hs, wh_ref) + lb_ref[...]
            i_ = jax.nn.sigmoid(gg[:, 0:D])
            f_ = jax.nn.sigmoid(gg[:, D:2 * D])
            g_ = jnp.tanh(gg[:, 2 * D:3 * D])
            o_ = jax.nn.sigmoid(gg[:, 3 * D:4 * D])
            cs = f_ * cs + i_ * g_
            hs = o_ * jnp.tanh(cs)
            e2 = lax.dot_general(outv, hs, (((1,), (1,)), ((), ())),
                                 preferred_element_type=jnp.float32)
            em = jnp.where(mask, e2, -1e30)
            mseg = jnp.max(em, axis=0, keepdims=True)
            msegc = jnp.where(mseg < -1e29, 0.0, mseg)
            a = jnp.exp(em - msegc)
            den = jnp.maximum(jnp.sum(a, axis=0, keepdims=True), 1e-16)
            an = a / den
            rvec = lax.dot_general(an, outv, (((0,), (0,)), ((), ())),
                                   preferred_element_type=jnp.float32)
            q_star = jnp.concatenate([hs, rvec], axis=1)
        pooled = jnp.maximum(_dot_t(q_star, l1_ref) + l1b_ref[...], 0.0)
        cat = jnp.concatenate([pooled, out_fp], axis=1)
        res_ref[...] = _dot_t(cat, l2_ref) + l2b_ref[...]

    return pl.pallas_call(
        body,
        out_shape=jax.ShapeDtypeStruct((g, odim), jnp.float32),
    )(out, bcol, fp, fc1W, fc1b, bng, bnb, lstmWi, lstmWh, lstmb,
      lin1W, lin1b, lin2W, lin2b)


def kernel(x, fp, edge_attr, params, edge_index, batch):
    p = params
    n = x.shape[0]
    src = edge_index[0]
    dst = edge_index[1]
    zeros = jnp.zeros((n, D), jnp.float32)
    onecol = jnp.zeros((CH, D), jnp.float32).at[:, 0].set(1.0)
    eye = jnp.eye(D, dtype=jnp.float32)
    emat = jnp.repeat(eye, D, axis=1)
    smat = jnp.tile(eye, (4, 1))
    w2_bf = p['enn_W2'].astype(jnp.bfloat16)
    emat_bf = emat.astype(jnp.bfloat16)

    degp = _sc_scatter_add(None, dst, zeros, const_rows=onecol)
    out, rdeg = _tc_prep(x, p['lin0_W'], p['lin0_b'].reshape(1, -1), degp)
    for _ in range(3):
        t = _sc_gather(out, src)
        msg = _tc_msg(edge_attr, t, p['enn_W1'], p['enn_b1'].reshape(1, -1),
                      w2_bf, emat_bf, smat)
        aggp = _sc_scatter_add(msg, dst, zeros)
        out = _tc_update(out, aggp, rdeg, p['root_W'],
                         p['conv_b'].reshape(1, -1), p['gru_Wi'], p['gru_Wh'],
                         p['gru_bi'].reshape(1, -1), p['gru_bh'].reshape(1, -1))
    lstmb = (p['lstm_bi'] + p['lstm_bh']).reshape(1, -1)
    return _tc_final(out, batch.reshape(-1, 1), fp, p['fc1_W'],
                     p['fc1_b'].reshape(1, -1), p['bn1_g'].reshape(1, -1),
                     p['bn1_b'].reshape(1, -1), p['lstm_Wi'], p['lstm_Wh'],
                     lstmb, p['lin1_W'], p['lin1_b'].reshape(1, -1),
                     p['lin2_W'], p['lin2_b'].reshape(1, -1))


# speedup vs baseline: 1.3090x; 1.1016x over previous
import functools

import jax
import jax.numpy as jnp
from jax import lax
from jax.experimental import pallas as pl
from jax.experimental.pallas import tpu as pltpu
from jax.experimental.pallas import tpu_sc as plsc

D = 32
NC, NS = 2, 16
NW = NC * NS
CH = 128

_MESH = dict(core_axis_name="c", subcore_axis_name="s")
_SC_PARAMS = pltpu.CompilerParams(use_tc_tiling_on_sc=False)


def _slabs(n):
    base = ((n // NS) // 8) * 8
    slabs = [base] * NS
    slabs[-1] = n - base * (NS - 1)
    starts = [base * i for i in range(NS)]
    return starts, slabs


def _sc_gather(table, idx):
    n, d = table.shape
    e = idx.shape[0]
    nch = e // CH
    nfull, rem = nch // NW, nch % NW
    mesh = plsc.VectorSubcoreMesh(**_MESH)

    @functools.partial(
        pl.kernel,
        out_type=jax.ShapeDtypeStruct((e, d), jnp.float32),
        mesh=mesh,
        compiler_params=_SC_PARAMS,
        scratch_types=[
            pltpu.VMEM((nfull + 1, CH), jnp.int32),
            pltpu.VMEM((2, CH, d), jnp.float32),
            pltpu.SemaphoreType.DMA,
            pltpu.SemaphoreType.DMA,
            pltpu.SemaphoreType.DMA,
        ],
    )
    def k(table_hbm, idx_hbm, out_hbm, idxb, rows, sem_i, sem_g0, sem_g1):
        cid = lax.axis_index("c")
        sid = lax.axis_index("s")
        w = sid * NC + cid
        nloc = jnp.where(w < rem, nfull + 1, nfull)

        def fire(i, carry):
            pltpu.async_copy(idx_hbm.at[pl.ds((w + i * NW) * CH, CH)],
                             idxb.at[i], sem_i)
            return carry

        def drain(i, carry):
            pltpu.make_async_copy(idx_hbm.at[pl.ds((w + i * NW) * CH, CH)],
                                  idxb.at[i], sem_i).wait()
            return carry

        lax.fori_loop(0, nloc, fire, 0)
        lax.fori_loop(0, nloc, drain, 0)

        pltpu.async_copy(table_hbm.at[idxb.at[0]], rows.at[0], sem_g0)

        def body(i, carry):
            def step(b, semg_b, semg_nb):
                pltpu.make_async_copy(table_hbm.at[idxb.at[i]], rows.at[b],
                                      semg_b).wait()

                @pl.when(i + 1 < nloc)
                def _():
                    pltpu.async_copy(table_hbm.at[idxb.at[i + 1]],
                                     rows.at[1 - b], semg_nb)

                pltpu.sync_copy(rows.at[b],
                                out_hbm.at[pl.ds((w + i * NW) * CH, CH)])

            @pl.when(lax.rem(i, 2) == 0)
            def _():
                step(0, sem_g0, sem_g1)

            @pl.when(lax.rem(i, 2) == 1)
            def _():
                step(1, sem_g1, sem_g0)

            return carry

        lax.fori_loop(0, nloc, body, 0)

    return k(table, idx)


def _sc_scatter_add(vals, idx, zeros, const_rows=None):
    n = zeros.shape[0]
    e = idx.shape[0]
    nch = e // CH
    nfull, rem = nch // NW, nch % NW
    mesh = plsc.VectorSubcoreMesh(**_MESH)
    starts, sizes = _slabs(n)
    use_const = const_rows is not None
    ins = ((idx, zeros, const_rows) if use_const else (vals, idx, zeros))

    @functools.partial(
        pl.kernel,
        out_type=jax.ShapeDtypeStruct((NC, n, D), jnp.float32),
        mesh=mesh,
        compiler_params=_SC_PARAMS,
        scratch_types=[
            pltpu.VMEM((nfull + 1, CH), jnp.int32),
            pltpu.VMEM((2, CH, D), jnp.float32),
            pltpu.VMEM_SHARED((n, D), jnp.float32),
            pltpu.SemaphoreType.DMA,
            pltpu.SemaphoreType.DMA,
            pltpu.SemaphoreType.DMA,
        ],
    )
    def k(*refs):
        if use_const:
            (idx_hbm, zeros_hbm, const_hbm, out_hbm,
             idxb, vbuf, acc, sem_i, sem_v0, sem_v1) = refs
            vals_hbm = None
        else:
            (vals_hbm, idx_hbm, zeros_hbm, out_hbm,
             idxb, vbuf, acc, sem_i, sem_v0, sem_v1) = refs
        cid = lax.axis_index("c")
        sid = lax.axis_index("s")
        w = sid * NC + cid
        nloc = jnp.where(w < rem, nfull + 1, nfull)

        def fire(i, carry):
            pltpu.async_copy(idx_hbm.at[pl.ds((w + i * NW) * CH, CH)],
                             idxb.at[i], sem_i)
            return carry

        lax.fori_loop(0, nloc, fire, 0)

        for s in range(NS):
            @pl.when(sid == s)
            def _():
                pltpu.sync_copy(zeros_hbm.at[pl.ds(starts[s], sizes[s])],
                                acc.at[pl.ds(starts[s], sizes[s])])
        if use_const:
            pltpu.sync_copy(const_hbm, vbuf.at[0])

        def drain(i, carry):
            pltpu.make_async_copy(idx_hbm.at[pl.ds((w + i * NW) * CH, CH)],
                                  idxb.at[i], sem_i).wait()
            return carry

        lax.fori_loop(0, nloc, drain, 0)
        plsc.subcore_barrier()

        if use_const:
            def body(i, carry):
                pltpu.sync_copy(vbuf.at[0], acc.at[idxb.at[i]], add=True)
                return carry

            lax.fori_loop(0, nloc, body, 0)
        else:
            pltpu.async_copy(vals_hbm.at[pl.ds(w * CH, CH)], vbuf.at[0],
                             sem_v0)

            def body(i, carry):
                def step(b, semv_b, semv_nb):
                    pltpu.make_async_copy(
                        vals_hbm.at[pl.ds((w + i * NW) * CH, CH)],
                        vbuf.at[b], semv_b).wait()

                    @pl.when(i + 1 < nloc)
                    def _():
                        pltpu.async_copy(
                            vals_hbm.at[pl.ds((w + (i + 1) * NW) * CH, CH)],
                            vbuf.at[1 - b], semv_nb)

                    pltpu.sync_copy(vbuf.at[b], acc.at[idxb.at[i]], add=True)

                @pl.when(lax.rem(i, 2) == 0)
                def _():
                    step(0, sem_v0, sem_v1)

                @pl.when(lax.rem(i, 2) == 1)
                def _():
                    step(1, sem_v1, sem_v0)

                return carry

            lax.fori_loop(0, nloc, body, 0)
        plsc.subcore_barrier()

        for s in range(NS):
            @pl.when(sid == s)
            def _():
                pltpu.sync_copy(acc.at[pl.ds(starts[s], sizes[s])],
                                out_hbm.at[cid, pl.ds(starts[s], sizes[s])])

    return k(*ins)


def _dot_t(a, b_ref):
    return lax.dot_general(a, b_ref[...], (((1,), (1,)), ((), ())),
                           preferred_element_type=jnp.float32)


def _tc_prep(x, lin0W, lin0b, degp):
    n = x.shape[0]

    def body(x_ref, w_ref, b_ref, degp_ref, out_ref, rdeg_ref):
        out_ref[...] = jnp.maximum(_dot_t(x_ref[...], w_ref) + b_ref[...], 0.0)
        dp = degp_ref[...]
        deg = dp[0, :, 0:1] + dp[1, :, 0:1]
        rdeg_ref[...] = 1.0 / jnp.maximum(deg, 1.0)

    return pl.pallas_call(
        body,
        out_shape=(jax.ShapeDtypeStruct((n, D), jnp.float32),
                   jax.ShapeDtypeStruct((n, 1), jnp.float32)),
    )(x, lin0W, lin0b, degp)


def _tc_msg(edge_attr, t, W1, b1, W2, Emat, Smat):
    e = edge_attr.shape[0]
    EB = 1000
    grid = e // EB

    def body(ea_ref, t_ref, w1_ref, b1_ref, w2_ref, em_ref, sm_ref, msg_ref):
        eh = jnp.maximum(_dot_t(ea_ref[...], w1_ref) + b1_ref[...], 0.0)
        w = lax.dot_general(eh.astype(jnp.bfloat16), w2_ref[...],
                            (((1,), (1,)), ((), ())),
                            preferred_element_type=jnp.float32)
        texp = jnp.dot(t_ref[...].astype(jnp.bfloat16), em_ref[...],
                       preferred_element_type=jnp.float32)
        p = texp * w
        p = p[:, :512] + p[:, 512:]
        p = p[:, :256] + p[:, 256:]
        p = p[:, :128] + p[:, 128:]
        msg_ref[...] = jnp.dot(p, sm_ref[...],
                               preferred_element_type=jnp.float32)

    return pl.pallas_call(
        body,
        grid=(grid,),
        in_specs=[
            pl.BlockSpec((EB, 4), lambda i: (i, 0)),
            pl.BlockSpec((EB, D), lambda i: (i, 0)),
            pl.BlockSpec((128, 4), lambda i: (0, 0)),
            pl.BlockSpec((1, 128), lambda i: (0, 0)),
            pl.BlockSpec((D * D, 128), lambda i: (0, 0)),
            pl.BlockSpec((D, D * D), lambda i: (0, 0)),
            pl.BlockSpec((4 * D, D), lambda i: (0, 0)),
        ],
        out_specs=pl.BlockSpec((EB, D), lambda i: (i, 0)),
        out_shape=jax.ShapeDtypeStruct((e, D), jnp.float32),
    )(edge_attr, t, W1, b1, W2, Emat, Smat)


def _tc_update(h, aggp, rdeg, rootW, convb, gruWi, gruWh, gbi, gbh):
    n = h.shape[0]

    def body(h_ref, aggp_ref, rdeg_ref, rw_ref, cb_ref, wi_ref, wh_ref,
             bi_ref, bh_ref, out_ref):
        hv = h_ref[...]
        ap = aggp_ref[...]
        agg = (ap[0] + ap[1]) * rdeg_ref[...]
        m = jnp.maximum(jnp.dot(hv, rw_ref[...],
                                preferred_element_type=jnp.float32)
                        + agg + cb_ref[...], 0.0)
        gi = _dot_t(m, wi_ref) + bi_ref[...]
        gh = _dot_t(hv, wh_ref) + bh_ref[...]
        r = jax.nn.sigmoid(gi[:, 0:D] + gh[:, 0:D])
        z = jax.nn.sigmoid(gi[:, D:2 * D] + gh[:, D:2 * D])
        nn = jnp.tanh(gi[:, 2 * D:3 * D] + r * gh[:, 2 * D:3 * D])
        out_ref[...] = (1.0 - z) * nn + z * hv

    return pl.pallas_call(
        body,
        out_shape=jax.ShapeDtypeStruct((n, D), jnp.float32),
    )(h, aggp, rdeg, rootW, convb, gruWi, gruWh, gbi, gbh)


def _tc_final(out, bcol, fp, fc1W, fc1b, bng, bnb, lstmWi, lstmWh, lstmb,
              lin1W, lin1b, lin2W, lin2b):
    n = out.shape[0]
    g = fp.shape[0]
    odim = lin2W.shape[0]

    def body(out_ref, bcol_ref, fp_ref, fc1_ref, fc1b_ref, bng_ref, bnb_ref,
             wi_ref, wh_ref, lb_ref, l1_ref, l1b_ref, l2_ref, l2b_ref,
             res_ref):
        outv = out_ref[...]
        bc = bcol_ref[...]
        gids = lax.broadcasted_iota(jnp.int32, (n, g), 1)
        mask = bc == gids
        hfp = _dot_t(fp_ref[...], fc1_ref) + fc1b_ref[...]
        hfp = hfp * (bng_ref[...] / jnp.sqrt(1.0 + 1e-5)) + bnb_ref[...]
        out_fp = jnp.where(hfp > 0.0, hfp,
                           jnp.exp(jnp.minimum(hfp, 0.0)) - 1.0)
        q_star = jnp.zeros((g, 2 * D), jnp.float32)
        hs = jnp.zeros((g, D), jnp.float32)
        cs = jnp.zeros((g, D), jnp.float32)
        for _ in range(3):
            gg = _dot_t(q_star, wi_ref) + _dot_t(hs, wh_ref) + lb_ref[...]
            i_ = jax.nn.sigmoid(gg[:, 0:D])
            f_ = jax.nn.sigmoid(gg[:, D:2 * D])
            g_ = jnp.tanh(gg[:, 2 * D:3 * D])
            o_ = jax.nn.sigmoid(gg[:, 3 * D:4 * D])
            cs = f_ * cs + i_ * g_
            hs = o_ * jnp.tanh(cs)
            e2 = lax.dot_general(outv, hs, (((1,), (1,)), ((), ())),
                                 preferred_element_type=jnp.float32)
            em = jnp.where(mask, e2, -1e30)
            mseg = jnp.max(em, axis=0, keepdims=True)
            msegc = jnp.where(mseg < -1e29, 0.0, mseg)
            a = jnp.exp(em - msegc)
            den = jnp.maximum(jnp.sum(a, axis=0, keepdims=True), 1e-16)
            an = a / den
            rvec = lax.dot_general(an, outv, (((0,), (0,)), ((), ())),
                                   preferred_element_type=jnp.float32)
            q_star = jnp.concatenate([hs, rvec], axis=1)
        pooled = jnp.maximum(_dot_t(q_star, l1_ref) + l1b_ref[...], 0.0)
        cat = jnp.concatenate([pooled, out_fp], axis=1)
        res_ref[...] = _dot_t(cat, l2_ref) + l2b_ref[...]

    return pl.pallas_call(
        body,
        out_shape=jax.ShapeDtypeStruct((g, odim), jnp.float32),
    )(out, bcol, fp, fc1W, fc1b, bng, bnb, lstmWi, lstmWh, lstmb,
      lin1W, lin1b, lin2W, lin2b)


def kernel(x, fp, edge_attr, params, edge_index, batch):
    p = params
    n = x.shape[0]
    src = edge_index[0]
    dst = edge_index[1]
    zeros = jnp.zeros((n, D), jnp.float32)
    onecol = jnp.zeros((CH, D), jnp.float32).at[:, 0].set(1.0)
    eye = jnp.eye(D, dtype=jnp.float32)
    emat = jnp.repeat(eye, D, axis=1)
    smat = jnp.tile(eye, (4, 1))
    w2_bf = p['enn_W2'].astype(jnp.bfloat16)
    emat_bf = emat.astype(jnp.bfloat16)

    degp = _sc_scatter_add(None, dst, zeros, const_rows=onecol)
    out, rdeg = _tc_prep(x, p['lin0_W'], p['lin0_b'].reshape(1, -1), degp)
    for _ in range(3):
        t = _sc_gather(out, src)
        msg = _tc_msg(edge_attr, t, p['enn_W1'], p['enn_b1'].reshape(1, -1),
                      w2_bf, emat_bf, smat)
        aggp = _sc_scatter_add(msg, dst, zeros)
        out = _tc_update(out, aggp, rdeg, p['root_W'],
                         p['conv_b'].reshape(1, -1), p['gru_Wi'], p['gru_Wh'],
                         p['gru_bi'].reshape(1, -1), p['gru_bh'].reshape(1, -1))
    lstmb = (p['lstm_bi'] + p['lstm_bh']).reshape(1, -1)
    return _tc_final(out, batch.reshape(-1, 1), fp, p['fc1_W'],
                     p['fc1_b'].reshape(1, -1), p['bn1_g'].reshape(1, -1),
                     p['bn1_b'].reshape(1, -1), p['lstm_Wi'], p['lstm_Wh'],
                     lstmb, p['lin1_W'], p['lin1_b'].reshape(1, -1),
                     p['lin2_W'], p['lin2_b'].reshape(1, -1))


# speedup vs baseline: 1.8440x; 1.4087x over previous
import functools

import jax
import jax.numpy as jnp
from jax import lax
from jax.experimental import pallas as pl
from jax.experimental.pallas import tpu as pltpu
from jax.experimental.pallas import tpu_sc as plsc

D = 32
NC, NS = 2, 16
NW = NC * NS
CH = 128

_MESH = dict(core_axis_name="c", subcore_axis_name="s")
_SC_PARAMS = pltpu.CompilerParams(use_tc_tiling_on_sc=False)


def _slabs(n):
    base = ((n // NS) // 8) * 8
    slabs = [base] * NS
    slabs[-1] = n - base * (NS - 1)
    starts = [base * i for i in range(NS)]
    return starts, slabs


def _sc_gather(table, idx):
    n, d = table.shape
    e = idx.shape[0]
    nch = e // CH
    nfull, rem = nch // NW, nch % NW
    mesh = plsc.VectorSubcoreMesh(**_MESH)

    @functools.partial(
        pl.kernel,
        out_type=jax.ShapeDtypeStruct((e, 128), jnp.float32),
        mesh=mesh,
        compiler_params=_SC_PARAMS,
        scratch_types=[
            pltpu.VMEM((nfull + 1, CH), jnp.int32),
            pltpu.VMEM((2, CH, d), jnp.float32),
            pltpu.SemaphoreType.DMA,
            pltpu.SemaphoreType.DMA,
            pltpu.SemaphoreType.DMA,
        ],
    )
    def k(table_hbm, idx_hbm, out_hbm, idxb, rows, sem_i, sem_g0, sem_g1):
        cid = lax.axis_index("c")
        sid = lax.axis_index("s")
        w = sid * NC + cid
        nloc = jnp.where(w < rem, nfull + 1, nfull)

        def fire(i, carry):
            pltpu.async_copy(idx_hbm.at[pl.ds((w + i * NW) * CH, CH)],
                             idxb.at[i], sem_i)
            return carry

        def drain(i, carry):
            pltpu.make_async_copy(idx_hbm.at[pl.ds((w + i * NW) * CH, CH)],
                                  idxb.at[i], sem_i).wait()
            return carry

        lax.fori_loop(0, nloc, fire, 0)
        lax.fori_loop(0, nloc, drain, 0)

        pltpu.async_copy(table_hbm.at[idxb.at[0]], rows.at[0], sem_g0)

        def body(i, carry):
            def step(b, semg_b, semg_nb):
                pltpu.make_async_copy(table_hbm.at[idxb.at[i]], rows.at[b],
                                      semg_b).wait()

                @pl.when(i + 1 < nloc)
                def _():
                    pltpu.async_copy(table_hbm.at[idxb.at[i + 1]],
                                     rows.at[1 - b], semg_nb)

                pltpu.sync_copy(
                    rows.at[b],
                    out_hbm.at[pl.ds((w + i * NW) * CH, CH), pl.ds(0, d)])

            @pl.when(lax.rem(i, 2) == 0)
            def _():
                step(0, sem_g0, sem_g1)

            @pl.when(lax.rem(i, 2) == 1)
            def _():
                step(1, sem_g1, sem_g0)

            return carry

        lax.fori_loop(0, nloc, body, 0)

    return k(table, idx)


def _sc_scatter_add(vals, idx, zeros, const_rows=None):
    n = zeros.shape[0]
    e = idx.shape[0]
    nch = e // CH
    nfull, rem = nch // NW, nch % NW
    mesh = plsc.VectorSubcoreMesh(**_MESH)
    starts, sizes = _slabs(n)
    use_const = const_rows is not None
    ins = ((idx, zeros, const_rows) if use_const else (vals, idx, zeros))

    @functools.partial(
        pl.kernel,
        out_type=jax.ShapeDtypeStruct((NC, n, D), jnp.float32),
        mesh=mesh,
        compiler_params=_SC_PARAMS,
        scratch_types=[
            pltpu.VMEM((nfull + 1, CH), jnp.int32),
            pltpu.VMEM((2, CH, D), jnp.float32),
            pltpu.VMEM_SHARED((n, D), jnp.float32),
            pltpu.SemaphoreType.DMA,
            pltpu.SemaphoreType.DMA,
            pltpu.SemaphoreType.DMA,
        ],
    )
    def k(*refs):
        if use_const:
            (idx_hbm, zeros_hbm, const_hbm, out_hbm,
             idxb, vbuf, acc, sem_i, sem_v0, sem_v1) = refs
            vals_hbm = None
        else:
            (vals_hbm, idx_hbm, zeros_hbm, out_hbm,
             idxb, vbuf, acc, sem_i, sem_v0, sem_v1) = refs
        cid = lax.axis_index("c")
        sid = lax.axis_index("s")
        w = sid * NC + cid
        nloc = jnp.where(w < rem, nfull + 1, nfull)

        def fire(i, carry):
            pltpu.async_copy(idx_hbm.at[pl.ds((w + i * NW) * CH, CH)],
                             idxb.at[i], sem_i)
            return carry

        lax.fori_loop(0, nloc, fire, 0)

        for s in range(NS):
            @pl.when(sid == s)
            def _():
                pltpu.sync_copy(zeros_hbm.at[pl.ds(starts[s], sizes[s])],
                                acc.at[pl.ds(starts[s], sizes[s])])
        if use_const:
            pltpu.sync_copy(const_hbm, vbuf.at[0])

        def drain(i, carry):
            pltpu.make_async_copy(idx_hbm.at[pl.ds((w + i * NW) * CH, CH)],
                                  idxb.at[i], sem_i).wait()
            return carry

        lax.fori_loop(0, nloc, drain, 0)
        plsc.subcore_barrier()

        if use_const:
            def body(i, carry):
                pltpu.sync_copy(vbuf.at[0], acc.at[idxb.at[i]], add=True)
                return carry

            lax.fori_loop(0, nloc, body, 0)
        else:
            pltpu.async_copy(vals_hbm.at[pl.ds(w * CH, CH), pl.ds(0, D)],
                             vbuf.at[0], sem_v0)

            def body(i, carry):
                def step(b, semv_b, semv_nb):
                    pltpu.make_async_copy(
                        vals_hbm.at[pl.ds((w + i * NW) * CH, CH),
                                    pl.ds(0, D)],
                        vbuf.at[b], semv_b).wait()

                    @pl.when(i + 1 < nloc)
                    def _():
                        pltpu.async_copy(
                            vals_hbm.at[pl.ds((w + (i + 1) * NW) * CH, CH),
                                        pl.ds(0, D)],
                            vbuf.at[1 - b], semv_nb)

                    pltpu.sync_copy(vbuf.at[b], acc.at[idxb.at[i]], add=True)

                @pl.when(lax.rem(i, 2) == 0)
                def _():
                    step(0, sem_v0, sem_v1)

                @pl.when(lax.rem(i, 2) == 1)
                def _():
                    step(1, sem_v1, sem_v0)

                return carry

            lax.fori_loop(0, nloc, body, 0)
        plsc.subcore_barrier()

        for s in range(NS):
            @pl.when(sid == s)
            def _():
                pltpu.sync_copy(acc.at[pl.ds(starts[s], sizes[s])],
                                out_hbm.at[cid, pl.ds(starts[s], sizes[s])])

    return k(*ins)


def _dot_t(a, b_ref):
    return lax.dot_general(a, b_ref[...], (((1,), (1,)), ((), ())),
                           preferred_element_type=jnp.float32)


def _tc_prep(x, lin0W, lin0b, degp):
    n = x.shape[0]

    def body(x_ref, w_ref, b_ref, degp_ref, out_ref, rdeg_ref):
        out_ref[...] = jnp.maximum(_dot_t(x_ref[...], w_ref) + b_ref[...], 0.0)
        dp = degp_ref[...]
        deg = dp[0, :, 0:1] + dp[1, :, 0:1]
        rdeg_ref[...] = 1.0 / jnp.maximum(deg, 1.0)

    return pl.pallas_call(
        body,
        out_shape=(jax.ShapeDtypeStruct((n, D), jnp.float32),
                   jax.ShapeDtypeStruct((n, 1), jnp.float32)),
    )(x, lin0W, lin0b, degp)


def _tc_msg(edge_attr, t, W1, b1, W2, Emat, Smat):
    e = edge_attr.shape[0]
    EB = 1600
    grid = e // EB

    def body(ea_ref, t_ref, w1_ref, b1_ref, w2_ref, em_ref, sm_ref, msg_ref):
        t = t_ref[...][:, 0:D]
        eh = jnp.maximum(_dot_t(ea_ref[...], w1_ref) + b1_ref[...], 0.0)
        w = lax.dot_general(eh.astype(jnp.bfloat16), w2_ref[...],
                            (((1,), (1,)), ((), ())),
                            preferred_element_type=jnp.float32)
        texp = jnp.dot(t.astype(jnp.bfloat16), em_ref[...],
                       preferred_element_type=jnp.float32)
        p = texp * w
        p = p[:, :512] + p[:, 512:]
        p = p[:, :256] + p[:, 256:]
        p = p[:, :128] + p[:, 128:]
        msg_ref[:, 0:D] = jnp.dot(p, sm_ref[...],
                                  preferred_element_type=jnp.float32)

    return pl.pallas_call(
        body,
        grid=(grid,),
        in_specs=[
            pl.BlockSpec((EB, 4), lambda i: (i, 0)),
            pl.BlockSpec((EB, 128), lambda i: (i, 0)),
            pl.BlockSpec((128, 4), lambda i: (0, 0)),
            pl.BlockSpec((1, 128), lambda i: (0, 0)),
            pl.BlockSpec((D * D, 128), lambda i: (0, 0)),
            pl.BlockSpec((D, D * D), lambda i: (0, 0)),
            pl.BlockSpec((4 * D, D), lambda i: (0, 0)),
        ],
        out_specs=pl.BlockSpec((EB, 128), lambda i: (i, 0)),
        out_shape=jax.ShapeDtypeStruct((e, 128), jnp.float32),
    )(edge_attr, t, W1, b1, W2, Emat, Smat)


def _tc_update(h, aggp, rdeg, rootW, convb, gruWi, gruWh, gbi, gbh):
    n = h.shape[0]

    def body(h_ref, aggp_ref, rdeg_ref, rw_ref, cb_ref, wi_ref, wh_ref,
             bi_ref, bh_ref, out_ref):
        hv = h_ref[...]
        ap = aggp_ref[...]
        agg = (ap[0] + ap[1]) * rdeg_ref[...]
        m = jnp.maximum(jnp.dot(hv, rw_ref[...],
                                preferred_element_type=jnp.float32)
                        + agg + cb_ref[...], 0.0)
        gi = _dot_t(m, wi_ref) + bi_ref[...]
        gh = _dot_t(hv, wh_ref) + bh_ref[...]
        r = jax.nn.sigmoid(gi[:, 0:D] + gh[:, 0:D])
        z = jax.nn.sigmoid(gi[:, D:2 * D] + gh[:, D:2 * D])
        nn = jnp.tanh(gi[:, 2 * D:3 * D] + r * gh[:, 2 * D:3 * D])
        out_ref[...] = (1.0 - z) * nn + z * hv

    return pl.pallas_call(
        body,
        out_shape=jax.ShapeDtypeStruct((n, D), jnp.float32),
    )(h, aggp, rdeg, rootW, convb, gruWi, gruWh, gbi, gbh)


def _tc_final(out, bcol, fp, fc1W, fc1b, bng, bnb, lstmWi, lstmWh, lstmb,
              lin1W, lin1b, lin2W, lin2b):
    n = out.shape[0]
    g = fp.shape[0]
    odim = lin2W.shape[0]

    def body(out_ref, bcol_ref, fp_ref, fc1_ref, fc1b_ref, bng_ref, bnb_ref,
             wi_ref, wh_ref, lb_ref, l1_ref, l1b_ref, l2_ref, l2b_ref,
             res_ref):
        outv = out_ref[...]
        bc = bcol_ref[...]
        gids = lax.broadcasted_iota(jnp.int32, (n, g), 1)
        mask = bc == gids
        hfp = _dot_t(fp_ref[...], fc1_ref) + fc1b_ref[...]
        hfp = hfp * (bng_ref[...] / jnp.sqrt(1.0 + 1e-5)) + bnb_ref[...]
        out_fp = jnp.where(hfp > 0.0, hfp,
                           jnp.exp(jnp.minimum(hfp, 0.0)) - 1.0)
        q_star = jnp.zeros((g, 2 * D), jnp.float32)
        hs = jnp.zeros((g, D), jnp.float32)
        cs = jnp.zeros((g, D), jnp.float32)
        for _ in range(3):
            gg = _dot_t(q_star, wi_ref) + _dot_t(hs, wh_ref) + lb_ref[...]
            i_ = jax.nn.sigmoid(gg[:, 0:D])
            f_ = jax.nn.sigmoid(gg[:, D:2 * D])
            g_ = jnp.tanh(gg[:, 2 * D:3 * D])
            o_ = jax.nn.sigmoid(gg[:, 3 * D:4 * D])
            cs = f_ * cs + i_ * g_
            hs = o_ * jnp.tanh(cs)
            e2 = lax.dot_general(outv, hs, (((1,), (1,)), ((), ())),
                                 preferred_element_type=jnp.float32)
            em = jnp.where(mask, e2, -1e30)
            mseg = jnp.max(em, axis=0, keepdims=True)
            msegc = jnp.where(mseg < -1e29, 0.0, mseg)
            a = jnp.exp(em - msegc)
            den = jnp.maximum(jnp.sum(a, axis=0, keepdims=True), 1e-16)
            an = a / den
            rvec = lax.dot_general(an, outv, (((0,), (0,)), ((), ())),
                                   preferred_element_type=jnp.float32)
            q_star = jnp.concatenate([hs, rvec], axis=1)
        pooled = jnp.maximum(_dot_t(q_star, l1_ref) + l1b_ref[...], 0.0)
        cat = jnp.concatenate([pooled, out_fp], axis=1)
        res_ref[...] = _dot_t(cat, l2_ref) + l2b_ref[...]

    return pl.pallas_call(
        body,
        out_shape=jax.ShapeDtypeStruct((g, odim), jnp.float32),
    )(out, bcol, fp, fc1W, fc1b, bng, bnb, lstmWi, lstmWh, lstmb,
      lin1W, lin1b, lin2W, lin2b)


def kernel(x, fp, edge_attr, params, edge_index, batch):
    p = params
    n = x.shape[0]
    src = edge_index[0]
    dst = edge_index[1]
    zeros = jnp.zeros((n, D), jnp.float32)
    onecol = jnp.zeros((CH, D), jnp.float32).at[:, 0].set(1.0)
    eye = jnp.eye(D, dtype=jnp.float32)
    emat = jnp.repeat(eye, D, axis=1)
    smat = jnp.tile(eye, (4, 1))
    w2_bf = p['enn_W2'].astype(jnp.bfloat16)
    emat_bf = emat.astype(jnp.bfloat16)

    degp = _sc_scatter_add(None, dst, zeros, const_rows=onecol)
    out, rdeg = _tc_prep(x, p['lin0_W'], p['lin0_b'].reshape(1, -1), degp)
    for _ in range(3):
        t128 = _sc_gather(out, src)
        msg128 = _tc_msg(edge_attr, t128, p['enn_W1'],
                         p['enn_b1'].reshape(1, -1), w2_bf, emat_bf, smat)
        aggp = _sc_scatter_add(msg128, dst, zeros)
        out = _tc_update(out, aggp, rdeg, p['root_W'],
                         p['conv_b'].reshape(1, -1), p['gru_Wi'], p['gru_Wh'],
                         p['gru_bi'].reshape(1, -1), p['gru_bh'].reshape(1, -1))
    lstmb = (p['lstm_bi'] + p['lstm_bh']).reshape(1, -1)
    return _tc_final(out, batch.reshape(-1, 1), fp, p['fc1_W'],
                     p['fc1_b'].reshape(1, -1), p['bn1_g'].reshape(1, -1),
                     p['bn1_b'].reshape(1, -1), p['lstm_Wi'], p['lstm_Wh'],
                     lstmb, p['lin1_W'], p['lin1_b'].reshape(1, -1),
                     p['lin2_W'], p['lin2_b'].reshape(1, -1))


# speedup vs baseline: 1.8657x; 1.0118x over previous
import functools

import jax
import jax.numpy as jnp
from jax import lax
from jax.experimental import pallas as pl
from jax.experimental.pallas import tpu as pltpu
from jax.experimental.pallas import tpu_sc as plsc

D = 32
NC, NS = 2, 16
NW = NC * NS
CH = 128

_MESH = dict(core_axis_name="c", subcore_axis_name="s")
_SC_PARAMS = pltpu.CompilerParams(use_tc_tiling_on_sc=False)


def _slabs(n):
    base = ((n // NS) // 8) * 8
    slabs = [base] * NS
    slabs[-1] = n - base * (NS - 1)
    starts = [base * i for i in range(NS)]
    return starts, slabs


def _sc_gather(table, idx):
    n, d = table.shape
    e = idx.shape[0]
    nch = e // CH
    nfull, rem = nch // NW, nch % NW
    mesh = plsc.VectorSubcoreMesh(**_MESH)

    @functools.partial(
        pl.kernel,
        out_type=jax.ShapeDtypeStruct((e, 128), jnp.float32),
        mesh=mesh,
        compiler_params=_SC_PARAMS,
        scratch_types=[
            pltpu.VMEM((nfull + 1, CH), jnp.int32),
            pltpu.VMEM((2, CH, d), jnp.float32),
            pltpu.SemaphoreType.DMA,
            pltpu.SemaphoreType.DMA,
            pltpu.SemaphoreType.DMA,
        ],
    )
    def k(table_hbm, idx_hbm, out_hbm, idxb, rows, sem_i, sem_g0, sem_g1):
        cid = lax.axis_index("c")
        sid = lax.axis_index("s")
        w = sid * NC + cid
        nloc = jnp.where(w < rem, nfull + 1, nfull)

        def fire(i, carry):
            pltpu.async_copy(idx_hbm.at[pl.ds((w + i * NW) * CH, CH)],
                             idxb.at[i], sem_i)
            return carry

        def drain(i, carry):
            pltpu.make_async_copy(idx_hbm.at[pl.ds((w + i * NW) * CH, CH)],
                                  idxb.at[i], sem_i).wait()
            return carry

        lax.fori_loop(0, nloc, fire, 0)
        lax.fori_loop(0, nloc, drain, 0)

        pltpu.async_copy(table_hbm.at[idxb.at[0]], rows.at[0], sem_g0)

        def body(i, carry):
            def step(b, semg_b, semg_nb):
                pltpu.make_async_copy(table_hbm.at[idxb.at[i]], rows.at[b],
                                      semg_b).wait()

                @pl.when(i + 1 < nloc)
                def _():
                    pltpu.async_copy(table_hbm.at[idxb.at[i + 1]],
                                     rows.at[1 - b], semg_nb)

                pltpu.sync_copy(
                    rows.at[b],
                    out_hbm.at[pl.ds((w + i * NW) * CH, CH), pl.ds(0, d)])

            @pl.when(lax.rem(i, 2) == 0)
            def _():
                step(0, sem_g0, sem_g1)

            @pl.when(lax.rem(i, 2) == 1)
            def _():
                step(1, sem_g1, sem_g0)

            return carry

        lax.fori_loop(0, nloc, body, 0)

    return k(table, idx)


def _sc_scatter_add(vals, idx, zeros, const_rows=None):
    n = zeros.shape[0]
    e = idx.shape[0]
    nch = e // CH
    nfull, rem = nch // NW, nch % NW
    mesh = plsc.VectorSubcoreMesh(**_MESH)
    starts, sizes = _slabs(n)
    use_const = const_rows is not None
    ins = ((idx, zeros, const_rows) if use_const else (vals, idx, zeros))

    @functools.partial(
        pl.kernel,
        out_type=jax.ShapeDtypeStruct((NC, n, 128), jnp.float32),
        mesh=mesh,
        compiler_params=_SC_PARAMS,
        scratch_types=[
            pltpu.VMEM((nfull + 1, CH), jnp.int32),
            pltpu.VMEM((2, CH, D), jnp.float32),
            pltpu.VMEM_SHARED((n, D), jnp.float32),
            pltpu.SemaphoreType.DMA,
            pltpu.SemaphoreType.DMA,
            pltpu.SemaphoreType.DMA,
        ],
    )
    def k(*refs):
        if use_const:
            (idx_hbm, zeros_hbm, const_hbm, out_hbm,
             idxb, vbuf, acc, sem_i, sem_v0, sem_v1) = refs
            vals_hbm = None
        else:
            (vals_hbm, idx_hbm, zeros_hbm, out_hbm,
             idxb, vbuf, acc, sem_i, sem_v0, sem_v1) = refs
        cid = lax.axis_index("c")
        sid = lax.axis_index("s")
        w = sid * NC + cid
        nloc = jnp.where(w < rem, nfull + 1, nfull)

        def fire(i, carry):
            pltpu.async_copy(idx_hbm.at[pl.ds((w + i * NW) * CH, CH)],
                             idxb.at[i], sem_i)
            return carry

        lax.fori_loop(0, nloc, fire, 0)

        for s in range(NS):
            @pl.when(sid == s)
            def _():
                pltpu.sync_copy(
                    zeros_hbm.at[pl.ds(starts[s], sizes[s]), pl.ds(0, D)],
                    acc.at[pl.ds(starts[s], sizes[s])])
        if use_const:
            pltpu.sync_copy(const_hbm, vbuf.at[0])

        def drain(i, carry):
            pltpu.make_async_copy(idx_hbm.at[pl.ds((w + i * NW) * CH, CH)],
                                  idxb.at[i], sem_i).wait()
            return carry

        lax.fori_loop(0, nloc, drain, 0)
        plsc.subcore_barrier()

        if use_const:
            def body(i, carry):
                pltpu.sync_copy(vbuf.at[0], acc.at[idxb.at[i]], add=True)
                return carry

            lax.fori_loop(0, nloc, body, 0)
        else:
            pltpu.async_copy(vals_hbm.at[pl.ds(w * CH, CH), pl.ds(0, D)],
                             vbuf.at[0], sem_v0)

            def body(i, carry):
                def step(b, semv_b, semv_nb):
                    pltpu.make_async_copy(
                        vals_hbm.at[pl.ds((w + i * NW) * CH, CH),
                                    pl.ds(0, D)],
                        vbuf.at[b], semv_b).wait()

                    @pl.when(i + 1 < nloc)
                    def _():
                        pltpu.async_copy(
                            vals_hbm.at[pl.ds((w + (i + 1) * NW) * CH, CH),
                                        pl.ds(0, D)],
                            vbuf.at[1 - b], semv_nb)

                    pltpu.sync_copy(vbuf.at[b], acc.at[idxb.at[i]], add=True)

                @pl.when(lax.rem(i, 2) == 0)
                def _():
                    step(0, sem_v0, sem_v1)

                @pl.when(lax.rem(i, 2) == 1)
                def _():
                    step(1, sem_v1, sem_v0)

                return carry

            lax.fori_loop(0, nloc, body, 0)
        plsc.subcore_barrier()

        for s in range(NS):
            @pl.when(sid == s)
            def _():
                pltpu.sync_copy(
                    acc.at[pl.ds(starts[s], sizes[s])],
                    out_hbm.at[cid, pl.ds(starts[s], sizes[s]), pl.ds(0, D)])

    return k(*ins)


def _dot_t(a, b_ref):
    return lax.dot_general(a, b_ref[...], (((1,), (1,)), ((), ())),
                           preferred_element_type=jnp.float32)


def _tc_prep(x, lin0W, lin0b, degp):
    n = x.shape[0]

    def body(x_ref, w_ref, b_ref, degp_ref, out_ref, rdeg_ref):
        out_ref[...] = jnp.maximum(_dot_t(x_ref[...], w_ref) + b_ref[...], 0.0)
        dp = degp_ref[...]
        deg = dp[0][:, 0:1] + dp[1][:, 0:1]
        rdeg_ref[...] = 1.0 / jnp.maximum(deg, 1.0)

    return pl.pallas_call(
        body,
        out_shape=(jax.ShapeDtypeStruct((n, D), jnp.float32),
                   jax.ShapeDtypeStruct((n, 1), jnp.float32)),
    )(x, lin0W, lin0b, degp)


def _tc_msg(edge_attr, t, W1, b1, W2, Emat, Smat):
    e = edge_attr.shape[0]
    EB = 1600
    grid = e // EB

    def body(ea_ref, t_ref, w1_ref, b1_ref, w2_ref, em_ref, sm_ref, msg_ref):
        t = t_ref[...][:, 0:D]
        eh = jnp.maximum(_dot_t(ea_ref[...], w1_ref) + b1_ref[...], 0.0)
        w = lax.dot_general(eh.astype(jnp.bfloat16), w2_ref[...],
                            (((1,), (1,)), ((), ())),
                            preferred_element_type=jnp.float32)
        texp = jnp.dot(t.astype(jnp.bfloat16), em_ref[...],
                       preferred_element_type=jnp.float32)
        p = texp * w
        p = p[:, :512] + p[:, 512:]
        p = p[:, :256] + p[:, 256:]
        p = p[:, :128] + p[:, 128:]
        msg_ref[:, 0:D] = jnp.dot(p, sm_ref[...],
                                  preferred_element_type=jnp.float32)

    return pl.pallas_call(
        body,
        grid=(grid,),
        in_specs=[
            pl.BlockSpec((EB, 4), lambda i: (i, 0)),
            pl.BlockSpec((EB, 128), lambda i: (i, 0)),
            pl.BlockSpec((128, 4), lambda i: (0, 0)),
            pl.BlockSpec((1, 128), lambda i: (0, 0)),
            pl.BlockSpec((D * D, 128), lambda i: (0, 0)),
            pl.BlockSpec((D, D * D), lambda i: (0, 0)),
            pl.BlockSpec((4 * D, D), lambda i: (0, 0)),
        ],
        out_specs=pl.BlockSpec((EB, 128), lambda i: (i, 0)),
        out_shape=jax.ShapeDtypeStruct((e, 128), jnp.float32),
    )(edge_attr, t, W1, b1, W2, Emat, Smat)


def _tc_update(h, aggp, rdeg, rootW, convb, gruWi, gruWh, gbi, gbh):
    n = h.shape[0]

    def body(h_ref, aggp_ref, rdeg_ref, rw_ref, cb_ref, wi_ref, wh_ref,
             bi_ref, bh_ref, out_ref):
        hv = h_ref[...]
        ap = aggp_ref[...]
        agg = (ap[0][:, 0:D] + ap[1][:, 0:D]) * rdeg_ref[...]
        m = jnp.maximum(jnp.dot(hv, rw_ref[...],
                                preferred_element_type=jnp.float32)
                        + agg + cb_ref[...], 0.0)
        gi = _dot_t(m, wi_ref) + bi_ref[...]
        gh = _dot_t(hv, wh_ref) + bh_ref[...]
        r = jax.nn.sigmoid(gi[:, 0:D] + gh[:, 0:D])
        z = jax.nn.sigmoid(gi[:, D:2 * D] + gh[:, D:2 * D])
        nn = jnp.tanh(gi[:, 2 * D:3 * D] + r * gh[:, 2 * D:3 * D])
        out_ref[...] = (1.0 - z) * nn + z * hv

    return pl.pallas_call(
        body,
        out_shape=jax.ShapeDtypeStruct((n, D), jnp.float32),
    )(h, aggp, rdeg, rootW, convb, gruWi, gruWh, gbi, gbh)


def _tc_final(out, bcol, fp, fc1W, fc1b, bng, bnb, lstmWi, lstmWh, lstmb,
              lin1W, lin1b, lin2W, lin2b):
    n = out.shape[0]
    g = fp.shape[0]
    odim = lin2W.shape[0]

    def body(out_ref, bcol_ref, fp_ref, fc1_ref, fc1b_ref, bng_ref, bnb_ref,
             wi_ref, wh_ref, lb_ref, l1_ref, l1b_ref, l2_ref, l2b_ref,
             res_ref):
        outv = out_ref[...]
        bc = bcol_ref[...]
        gids = lax.broadcasted_iota(jnp.int32, (n, g), 1)
        mask = bc == gids
        hfp = _dot_t(fp_ref[...], fc1_ref) + fc1b_ref[...]
        hfp = hfp * (bng_ref[...] / jnp.sqrt(1.0 + 1e-5)) + bnb_ref[...]
        out_fp = jnp.where(hfp > 0.0, hfp,
                           jnp.exp(jnp.minimum(hfp, 0.0)) - 1.0)
        q_star = jnp.zeros((g, 2 * D), jnp.float32)
        hs = jnp.zeros((g, D), jnp.float32)
        cs = jnp.zeros((g, D), jnp.float32)
        for _ in range(3):
            gg = _dot_t(q_star, wi_ref) + _dot_t(hs, wh_ref) + lb_ref[...]
            i_ = jax.nn.sigmoid(gg[:, 0:D])
            f_ = jax.nn.sigmoid(gg[:, D:2 * D])
            g_ = jnp.tanh(gg[:, 2 * D:3 * D])
            o_ = jax.nn.sigmoid(gg[:, 3 * D:4 * D])
            cs = f_ * cs + i_ * g_
            hs = o_ * jnp.tanh(cs)
            e2 = lax.dot_general(outv, hs, (((1,), (1,)), ((), ())),
                                 preferred_element_type=jnp.float32)
            em = jnp.where(mask, e2, -1e30)
            mseg = jnp.max(em, axis=0, keepdims=True)
            msegc = jnp.where(mseg < -1e29, 0.0, mseg)
            a = jnp.exp(em - msegc)
            den = jnp.maximum(jnp.sum(a, axis=0, keepdims=True), 1e-16)
            an = a / den
            rvec = lax.dot_general(an, outv, (((0,), (0,)), ((), ())),
                                   preferred_element_type=jnp.float32)
            q_star = jnp.concatenate([hs, rvec], axis=1)
        pooled = jnp.maximum(_dot_t(q_star, l1_ref) + l1b_ref[...], 0.0)
        cat = jnp.concatenate([pooled, out_fp], axis=1)
        res_ref[...] = _dot_t(cat, l2_ref) + l2b_ref[...]

    return pl.pallas_call(
        body,
        out_shape=jax.ShapeDtypeStruct((g, odim), jnp.float32),
    )(out, bcol, fp, fc1W, fc1b, bng, bnb, lstmWi, lstmWh, lstmb,
      lin1W, lin1b, lin2W, lin2b)


def kernel(x, fp, edge_attr, params, edge_index, batch):
    p = params
    n = x.shape[0]
    src = edge_index[0]
    dst = edge_index[1]
    zeros = jnp.zeros((n, 128), jnp.float32)
    onecol = jnp.zeros((CH, D), jnp.float32).at[:, 0].set(1.0)
    eye = jnp.eye(D, dtype=jnp.float32)
    emat = jnp.repeat(eye, D, axis=1)
    smat = jnp.tile(eye, (4, 1))
    w2_bf = p['enn_W2'].astype(jnp.bfloat16)
    emat_bf = emat.astype(jnp.bfloat16)

    degp = _sc_scatter_add(None, dst, zeros, const_rows=onecol)
    out, rdeg = _tc_prep(x, p['lin0_W'], p['lin0_b'].reshape(1, -1), degp)
    for _ in range(3):
        t128 = _sc_gather(out, src)
        msg128 = _tc_msg(edge_attr, t128, p['enn_W1'],
                         p['enn_b1'].reshape(1, -1), w2_bf, emat_bf, smat)
        aggp = _sc_scatter_add(msg128, dst, zeros)
        out = _tc_update(out, aggp, rdeg, p['root_W'],
                         p['conv_b'].reshape(1, -1), p['gru_Wi'], p['gru_Wh'],
                         p['gru_bi'].reshape(1, -1), p['gru_bh'].reshape(1, -1))
    lstmb = (p['lstm_bi'] + p['lstm_bh']).reshape(1, -1)
    return _tc_final(out, batch.reshape(-1, 1), fp, p['fc1_W'],
                     p['fc1_b'].reshape(1, -1), p['bn1_g'].reshape(1, -1),
                     p['bn1_b'].reshape(1, -1), p['lstm_Wi'], p['lstm_Wh'],
                     lstmb, p['lin1_W'], p['lin1_b'].reshape(1, -1),
                     p['lin2_W'], p['lin2_b'].reshape(1, -1))
